# Initial kernel scaffold; baseline (speedup 1.0000x reference)
import jax
import jax.numpy as jnp
from jax.experimental import pallas as pl


def kernel(x, edge_index, W1, b1, W2, b2):
    raise NotImplementedError("write your pallas kernel here")


# speedup vs baseline: 7.0731x; 7.0731x over previous
import functools

import jax
import jax.numpy as jnp
from jax import lax
from jax.experimental import pallas as pl
from jax.experimental.pallas import tpu as pltpu
from jax.experimental.pallas import tpu_sc as plsc

N_NODES = 10000
N_EDGES = 320000
D_IN = 128
D_HID = 64

NUM_CORES = 2
NUM_SUBCORES = 16
NUM_TILES = NUM_CORES * NUM_SUBCORES

CHUNK = 128
NUM_CHUNKS = N_EDGES // CHUNK
MAX_ITERS = -(-NUM_CHUNKS // NUM_TILES)
N_PAD = 10240
ROWS_PER_TILE = N_PAD // NUM_SUBCORES
ZROWS = 128


@functools.lru_cache(maxsize=1)
def _make_aggregate():
  mesh = plsc.VectorSubcoreMesh(core_axis_name="c", subcore_axis_name="s",
                                num_cores=NUM_CORES,
                                num_subcores=NUM_SUBCORES)

  @functools.partial(
      pl.kernel,
      out_type=jax.ShapeDtypeStruct((NUM_CORES * N_PAD, D_HID), jnp.float32),
      mesh=mesh,
      scratch_types=[
          pltpu.VMEM((CHUNK,), jnp.int32),
          pltpu.VMEM((CHUNK,), jnp.int32),
          pltpu.VMEM((CHUNK, D_HID), jnp.float32),
          pltpu.VMEM((ZROWS, D_HID), jnp.float32),
          pltpu.VMEM_SHARED((N_PAD, D_HID), jnp.float32),
          pltpu.SemaphoreType.DMA,
      ],
      compiler_params=pltpu.CompilerParams(use_tc_tiling_on_sc=False),
  )
  def aggregate(y_hbm, src_hbm, dst_hbm, out_hbm,
                src_v, dst_v, rows_v, zero_v, acc, sem):
    c = lax.axis_index("c")
    s = lax.axis_index("s")
    w = c * NUM_SUBCORES + s

    def zrow(r, _):
      for q in range(D_HID // 16):
        zero_v[r, pl.ds(q * 16, 16)] = jnp.zeros((16,), jnp.float32)
      return 0
    lax.fori_loop(0, ZROWS, zrow, 0)
    row0 = s * ROWS_PER_TILE
    for k in range(ROWS_PER_TILE // ZROWS):
      pltpu.sync_copy(zero_v, acc.at[pl.ds(row0 + k * ZROWS, ZROWS), :])
    plsc.subcore_barrier()

    def body(i, _):
      chunk = w + i * NUM_TILES
      @pl.when(chunk < NUM_CHUNKS)
      def _():
        e0 = chunk * CHUNK
        pltpu.sync_copy(src_hbm.at[pl.ds(e0, CHUNK)], src_v)
        pltpu.sync_copy(dst_hbm.at[pl.ds(e0, CHUNK)], dst_v)
        pltpu.async_copy(y_hbm.at[src_v], rows_v, sem).wait()
        pltpu.sync_copy(rows_v, acc.at[dst_v], add=True)
      return 0
    lax.fori_loop(0, MAX_ITERS, body, 0)

    plsc.subcore_barrier()
    pltpu.sync_copy(acc.at[pl.ds(row0, ROWS_PER_TILE), :],
                    out_hbm.at[pl.ds(c * N_PAD + row0, ROWS_PER_TILE), :])

  return aggregate


def _aggregate(y, src, dst):
  return _make_aggregate()(y, src, dst)


_BLK = 1000
_GRID = N_NODES // _BLK


def _mm1_body(x_ref, w_ref, o_ref):
  o_ref[:, :] = jnp.dot(x_ref[:, :], w_ref[:, :],
                        preferred_element_type=jnp.float32)


def _matmul1(x, w1):
  return pl.pallas_call(
      _mm1_body,
      grid=(_GRID,),
      in_specs=[
          pl.BlockSpec((_BLK, D_IN), lambda i: (i, 0)),
          pl.BlockSpec((D_IN, D_HID), lambda i: (0, 0)),
      ],
      out_specs=pl.BlockSpec((_BLK, D_HID), lambda i: (i, 0)),
      out_shape=jax.ShapeDtypeStruct((N_NODES, D_HID), jnp.float32),
  )(x, w1)


def _relu_body(y_ref, pa_ref, pb_ref, b_ref, o_ref):
  o_ref[:, :] = jnp.maximum(
      y_ref[:, :] + pa_ref[:, :] + pb_ref[:, :] + b_ref[:, :], 0.0)


def _relu_sum(y, partials, b1):
  return pl.pallas_call(
      _relu_body,
      grid=(_GRID,),
      in_specs=[
          pl.BlockSpec((_BLK, D_HID), lambda i: (i, 0)),
          pl.BlockSpec((_BLK, D_HID), lambda i: (i, 0)),
          pl.BlockSpec((_BLK, D_HID), lambda i: (i, 0)),
          pl.BlockSpec((1, D_HID), lambda i: (0, 0)),
      ],
      out_specs=pl.BlockSpec((_BLK, D_HID), lambda i: (i, 0)),
      out_shape=jax.ShapeDtypeStruct((N_NODES, D_HID), jnp.float32),
  )(y, partials[:N_NODES], partials[N_PAD:N_PAD + N_NODES],
    b1.reshape(1, D_HID))


def _out_body(h_ref, qa_ref, qb_ref, w_ref, b_ref, o_ref):
  g = h_ref[:, :] + qa_ref[:, :] + qb_ref[:, :]
  o = jnp.dot(g, w_ref[:, :], preferred_element_type=jnp.float32) + b_ref[:, :]
  m = jnp.max(o, axis=1, keepdims=True)
  z = o - m
  o_ref[:, :] = z - jnp.log(jnp.sum(jnp.exp(z), axis=1, keepdims=True))


def _final(h, partials, w2, b2):
  return pl.pallas_call(
      _out_body,
      grid=(_GRID,),
      in_specs=[
          pl.BlockSpec((_BLK, D_HID), lambda i: (i, 0)),
          pl.BlockSpec((_BLK, D_HID), lambda i: (i, 0)),
          pl.BlockSpec((_BLK, D_HID), lambda i: (i, 0)),
          pl.BlockSpec((D_HID, D_IN), lambda i: (0, 0)),
          pl.BlockSpec((1, D_IN), lambda i: (0, 0)),
      ],
      out_specs=pl.BlockSpec((_BLK, D_IN), lambda i: (i, 0)),
      out_shape=jax.ShapeDtypeStruct((N_NODES, D_IN), jnp.float32),
  )(h, partials[:N_NODES], partials[N_PAD:N_PAD + N_NODES], w2,
    b2.reshape(1, D_IN))


def kernel(x, edge_index, W1, b1, W2, b2):
  src = edge_index[0]
  dst = edge_index[1]
  y1 = _matmul1(x, W1)
  p1 = _aggregate(y1, src, dst)
  h = _relu_sum(y1, p1, b1)
  p2 = _aggregate(h, src, dst)
  return _final(h, p2, W2, b2)
